# baseline (device time: 722981 ns/iter reference)
import jax
import jax.numpy as jnp
from jax import lax
from jax.experimental import pallas as pl
from jax.experimental.pallas import tpu as pltpu

N_DEV = 4
N_HOPS = N_DEV - 1
M_CH = 1024
K_SH = 1024
N_TOT = 8192
NC = 8
N_BLK = N_TOT // NC


def kernel(x, w_mat, scale_x, scale_w):
    xb = x.astype(jnp.bfloat16)
    wb = w_mat.astype(jnp.bfloat16)
    sc = (scale_x * scale_w).astype(jnp.float32).reshape(1, 1)

    def body(s_ref, x_ref, w_ref, o_ref,
             send_buf, recv_buf, send_sem, recv_sems, credit_sem):
        i = lax.axis_index("i")
        left = (i + N_DEV - 1) % N_DEV
        right = (i + 1) % N_DEV
        step = pl.program_id(0)
        barrier = pltpu.get_barrier_semaphore()

        @pl.when(step == 0)
        def _entry_barrier():
            for nbr in (left, right):
                pl.semaphore_signal(
                    barrier, inc=1, device_id=(nbr,),
                    device_id_type=pl.DeviceIdType.MESH,
                )
            pl.semaphore_wait(barrier, 2)

        def credit_to_left():
            pl.semaphore_signal(
                credit_sem, inc=1, device_id=(left,),
                device_id_type=pl.DeviceIdType.MESH,
            )

        for h in range(N_HOPS):
            j = (i + N_DEV - 1 - h) % N_DEV
            part = jnp.dot(
                x_ref[pl.ds(j * M_CH, M_CH), :], w_ref[:, :],
                preferred_element_type=jnp.float32,
            )
            if h == 0:
                acc = part
            else:
                acc = part + recv_buf[h - 1].astype(jnp.float32)
            send_buf[:, :] = acc.astype(jnp.bfloat16)
            if h > 0:
                @pl.when(step <= NC - 2)
                def _():
                    credit_to_left()

            @pl.when(step >= 1)
            def _():
                pl.semaphore_wait(credit_sem, 1)

            rdma = pltpu.make_async_remote_copy(
                src_ref=send_buf,
                dst_ref=recv_buf.at[h],
                send_sem=send_sem,
                recv_sem=recv_sems.at[h],
                device_id=(right,),
                device_id_type=pl.DeviceIdType.MESH,
            )
            rdma.start()
            rdma.wait()

        part = jnp.dot(
            x_ref[pl.ds(i * M_CH, M_CH), :], w_ref[:, :],
            preferred_element_type=jnp.float32,
        )
        acc = part + recv_buf[N_HOPS - 1].astype(jnp.float32)

        @pl.when(step <= NC - 2)
        def _():
            credit_to_left()

        y = acc * s_ref[0, 0]
        o_ref[:, :] = y / (1.0 + jnp.exp(-jnp.clip(y, -60.0, 60.0)))

    return pl.pallas_call(
        body,
        grid=(NC,),
        out_shape=jax.ShapeDtypeStruct((M_CH, N_TOT), jnp.float32),
        in_specs=[
            pl.BlockSpec(memory_space=pltpu.SMEM),
            pl.BlockSpec((N_DEV * M_CH, K_SH), lambda c: (0, 0)),
            pl.BlockSpec((K_SH, N_BLK), lambda c: (0, c)),
        ],
        out_specs=pl.BlockSpec((M_CH, N_BLK), lambda c: (0, c)),
        scratch_shapes=[
            pltpu.VMEM((M_CH, N_BLK), jnp.bfloat16),
            pltpu.VMEM((N_HOPS, M_CH, N_BLK), jnp.bfloat16),
            pltpu.SemaphoreType.DMA,
            pltpu.SemaphoreType.DMA((N_HOPS,)),
            pltpu.SemaphoreType.REGULAR,
        ],
        compiler_params=pltpu.CompilerParams(
            collective_id=0,
            dimension_semantics=("arbitrary",),
        ),
    )(sc, xb, wb)


# device time: 329744 ns/iter; 2.1926x vs baseline; 2.1926x over previous
import jax
import jax.numpy as jnp
from jax import lax
from jax.experimental import pallas as pl
from jax.experimental.pallas import tpu as pltpu

N_DEV = 4
N_HOPS = N_DEV - 1
M_CH = 1024
K_SH = 1024
N_TOT = 8192
NC = 8
N_BLK = N_TOT // NC
N_SUB = N_BLK // 4

PIPES = ((0, +1, 0), (1, +1, N_SUB), (2, -1, 2 * N_SUB), (3, -1, 3 * N_SUB))


def kernel(x, w_mat, scale_x, scale_w):
    xb = x.astype(jnp.float8_e4m3fn)
    wb = w_mat.astype(jnp.float8_e5m2)
    sc = (scale_x * scale_w).astype(jnp.float32).reshape(1, 1)

    def body(s_ref, x_any, wc_ref, wn_ref, o_ref,
             x_vmem, send_buf, recv_buf,
             copy_sem, send_sems, recv_sems, credit_sems):
        i = lax.axis_index("i")
        left = (i + N_DEV - 1) % N_DEV
        right = (i + 1) % N_DEV
        step = pl.program_id(0)
        barrier = pltpu.get_barrier_semaphore()

        def jchunk(dirn, h):
            if dirn > 0:
                return (i + N_DEV - 1 - h) % N_DEV
            return (i + 1 + h) % N_DEV

        def rdma(p, dirn, h):
            dst = right if dirn > 0 else left
            return pltpu.make_async_remote_copy(
                src_ref=send_buf.at[p, h],
                dst_ref=recv_buf.at[p, h],
                send_sem=send_sems.at[p, h],
                recv_sem=recv_sems.at[p, h],
                device_id=(dst,),
                device_id_type=pl.DeviceIdType.MESH,
            )

        def credit_to_sender(p, dirn):
            src = left if dirn > 0 else right
            pl.semaphore_signal(
                credit_sems.at[p], inc=1, device_id=(src,),
                device_id_type=pl.DeviceIdType.MESH,
            )

        def dot_rows(j, w_ref, off):
            return jnp.dot(
                x_vmem[pl.ds(j * M_CH, M_CH), :],
                w_ref[:, off:off + N_SUB],
                preferred_element_type=jnp.float32,
            )

        @pl.when(step == 0)
        def _init():
            cx = pltpu.make_async_copy(x_any, x_vmem, copy_sem)
            cx.start()
            for nbr in (left, right):
                pl.semaphore_signal(
                    barrier, inc=1, device_id=(nbr,),
                    device_id_type=pl.DeviceIdType.MESH,
                )
            pl.semaphore_wait(barrier, 2)
            cx.wait()
            for p, dirn, off in PIPES:
                d0 = dot_rows(jchunk(dirn, 0), wc_ref, off)
                send_buf[p, 0] = d0.astype(jnp.bfloat16)
                rdma(p, dirn, 0).start()

        for h in (1, 2):
            for p, dirn, off in PIPES:
                dh = dot_rows(jchunk(dirn, h), wc_ref, off)
                rdma(p, dirn, h - 1).wait_recv()

                @pl.when(step <= NC - 2)
                def _credit():
                    credit_to_sender(p, dirn)

                acc = dh + recv_buf[p, h - 1].astype(jnp.float32)

                @pl.when(step >= 1)
                def _reuse():
                    rdma(p, dirn, h).wait_send()

                send_buf[p, h] = acc.astype(jnp.bfloat16)

                @pl.when(step >= 1)
                def _gate():
                    pl.semaphore_wait(credit_sems.at[p], 1)

                rdma(p, dirn, h).start()

        @pl.when(step <= NC - 2)
        def _next_hop0():
            for p, dirn, off in PIPES:
                d0 = dot_rows(jchunk(dirn, 0), wn_ref, off)
                rdma(p, dirn, 0).wait_send()
                send_buf[p, 0] = d0.astype(jnp.bfloat16)
                pl.semaphore_wait(credit_sems.at[p], 1)
                rdma(p, dirn, 0).start()

        for p, dirn, off in PIPES:
            di = dot_rows(i, wc_ref, off)
            rdma(p, dirn, 2).wait_recv()

            @pl.when(step <= NC - 2)
            def _credit2():
                credit_to_sender(p, dirn)

            y = (di + recv_buf[p, 2].astype(jnp.float32)) * s_ref[0, 0]
            o_ref[:, off:off + N_SUB] = y / (
                1.0 + jnp.exp(-jnp.clip(y, -60.0, 60.0))
            )

        @pl.when(step == NC - 1)
        def _drain():
            for p, dirn, off in PIPES:
                for h in range(N_HOPS):
                    rdma(p, dirn, h).wait_send()

    return pl.pallas_call(
        body,
        grid=(NC,),
        out_shape=jax.ShapeDtypeStruct((M_CH, N_TOT), jnp.float32),
        in_specs=[
            pl.BlockSpec(memory_space=pltpu.SMEM),
            pl.BlockSpec(memory_space=pl.ANY),
            pl.BlockSpec((K_SH, N_BLK), lambda c: (0, c)),
            pl.BlockSpec((K_SH, N_BLK), lambda c: (0, jnp.minimum(c + 1, NC - 1))),
        ],
        out_specs=pl.BlockSpec((M_CH, N_BLK), lambda c: (0, c)),
        scratch_shapes=[
            pltpu.VMEM((N_DEV * M_CH, K_SH), jnp.float8_e4m3fn),
            pltpu.VMEM((4, N_HOPS, M_CH, N_SUB), jnp.bfloat16),
            pltpu.VMEM((4, N_HOPS, M_CH, N_SUB), jnp.bfloat16),
            pltpu.SemaphoreType.DMA,
            pltpu.SemaphoreType.DMA((4, N_HOPS)),
            pltpu.SemaphoreType.DMA((4, N_HOPS)),
            pltpu.SemaphoreType.REGULAR((4,)),
        ],
        compiler_params=pltpu.CompilerParams(
            collective_id=0,
            dimension_semantics=("arbitrary",),
            vmem_limit_bytes=50 * 1024 * 1024,
        ),
    )(sc, xb, wb, wb)


# device time: 328880 ns/iter; 2.1983x vs baseline; 1.0026x over previous
import jax
import jax.numpy as jnp
from jax import lax
from jax.experimental import pallas as pl
from jax.experimental.pallas import tpu as pltpu

N_DEV = 4
N_HOPS = N_DEV - 1
M_CH = 1024
K_SH = 1024
N_TOT = 8192
NC = 8
N_BLK = N_TOT // NC
N_SUB = N_BLK // 4

PIPES = ((0, +1, 0), (1, -1, N_SUB), (2, +1, 2 * N_SUB), (3, -1, 3 * N_SUB))


def kernel(x, w_mat, scale_x, scale_w):
    xb = x.astype(jnp.float8_e4m3fn)
    wb = w_mat.astype(jnp.float8_e5m2)
    sc = (scale_x * scale_w).astype(jnp.float32).reshape(1, 1)

    def body(s_ref, x_any, wc_ref, wn_ref, o_ref,
             x_vmem, send_buf, recv_buf,
             copy_sem, send_sems, recv_sems, credit_sems):
        i = lax.axis_index("i")
        left = (i + N_DEV - 1) % N_DEV
        right = (i + 1) % N_DEV
        step = pl.program_id(0)
        barrier = pltpu.get_barrier_semaphore()

        def jchunk(dirn, h):
            if dirn > 0:
                return (i + N_DEV - 1 - h) % N_DEV
            return (i + 1 + h) % N_DEV

        def rdma(p, dirn, h):
            dst = right if dirn > 0 else left
            return pltpu.make_async_remote_copy(
                src_ref=send_buf.at[p, h],
                dst_ref=recv_buf.at[p, h],
                send_sem=send_sems.at[p, h],
                recv_sem=recv_sems.at[p, h],
                device_id=(dst,),
                device_id_type=pl.DeviceIdType.MESH,
            )

        def credit_to_sender(p, dirn):
            src = left if dirn > 0 else right
            pl.semaphore_signal(
                credit_sems.at[p], inc=1, device_id=(src,),
                device_id_type=pl.DeviceIdType.MESH,
            )

        def dot_rows(j, w_ref, off):
            return jnp.dot(
                x_vmem[pl.ds(j * M_CH, M_CH), :],
                w_ref[:, off:off + N_SUB],
                preferred_element_type=jnp.float32,
            )

        @pl.when(step == 0)
        def _init():
            cx = pltpu.make_async_copy(x_any, x_vmem, copy_sem)
            cx.start()
            for nbr in (left, right):
                pl.semaphore_signal(
                    barrier, inc=1, device_id=(nbr,),
                    device_id_type=pl.DeviceIdType.MESH,
                )
            pl.semaphore_wait(barrier, 2)
            cx.wait()
            for p, dirn, off in PIPES:
                d0 = dot_rows(jchunk(dirn, 0), wc_ref, off)
                send_buf[p, 0] = d0.astype(jnp.bfloat16)
                rdma(p, dirn, 0).start()

        for h in (1, 2):
            for p, dirn, off in PIPES:
                dh = dot_rows(jchunk(dirn, h), wc_ref, off)
                rdma(p, dirn, h - 1).wait_recv()

                @pl.when(step <= NC - 2)
                def _credit():
                    credit_to_sender(p, dirn)

                acc = dh + recv_buf[p, h - 1].astype(jnp.float32)

                @pl.when(step >= 1)
                def _reuse():
                    rdma(p, dirn, h).wait_send()

                send_buf[p, h] = acc.astype(jnp.bfloat16)

                @pl.when(step >= 1)
                def _gate():
                    pl.semaphore_wait(credit_sems.at[p], 1)

                rdma(p, dirn, h).start()

        @pl.when(step <= NC - 2)
        def _next_hop0():
            for p, dirn, off in PIPES:
                d0 = dot_rows(jchunk(dirn, 0), wn_ref, off)
                rdma(p, dirn, 0).wait_send()
                send_buf[p, 0] = d0.astype(jnp.bfloat16)
                pl.semaphore_wait(credit_sems.at[p], 1)
                rdma(p, dirn, 0).start()

        for p, dirn, off in PIPES:
            di = dot_rows(i, wc_ref, off)
            rdma(p, dirn, 2).wait_recv()

            @pl.when(step <= NC - 2)
            def _credit2():
                credit_to_sender(p, dirn)

            y = (di + recv_buf[p, 2].astype(jnp.float32)) * s_ref[0, 0]
            o_ref[:, off:off + N_SUB] = y / (
                1.0 + jnp.exp(-jnp.clip(y, -60.0, 60.0))
            )

        @pl.when(step == NC - 1)
        def _drain():
            for p, dirn, off in PIPES:
                for h in range(N_HOPS):
                    rdma(p, dirn, h).wait_send()

    return pl.pallas_call(
        body,
        grid=(NC,),
        out_shape=jax.ShapeDtypeStruct((M_CH, N_TOT), jnp.float32),
        in_specs=[
            pl.BlockSpec(memory_space=pltpu.SMEM),
            pl.BlockSpec(memory_space=pl.ANY),
            pl.BlockSpec((K_SH, N_BLK), lambda c: (0, c)),
            pl.BlockSpec((K_SH, N_BLK), lambda c: (0, jnp.minimum(c + 1, NC - 1))),
        ],
        out_specs=pl.BlockSpec((M_CH, N_BLK), lambda c: (0, c)),
        scratch_shapes=[
            pltpu.VMEM((N_DEV * M_CH, K_SH), jnp.float8_e4m3fn),
            pltpu.VMEM((4, N_HOPS, M_CH, N_SUB), jnp.bfloat16),
            pltpu.VMEM((4, N_HOPS, M_CH, N_SUB), jnp.bfloat16),
            pltpu.SemaphoreType.DMA,
            pltpu.SemaphoreType.DMA((4, N_HOPS)),
            pltpu.SemaphoreType.DMA((4, N_HOPS)),
            pltpu.SemaphoreType.REGULAR((4,)),
        ],
        compiler_params=pltpu.CompilerParams(
            collective_id=0,
            dimension_semantics=("arbitrary",),
            vmem_limit_bytes=50 * 1024 * 1024,
            allow_input_fusion=[False, False, True, True],
        ),
    )(sc, xb, wb, wb)


# device time: 317333 ns/iter; 2.2783x vs baseline; 1.0364x over previous
import jax
import jax.numpy as jnp
from jax import lax
from jax.experimental import pallas as pl
from jax.experimental.pallas import tpu as pltpu

N_DEV = 4
N_HOPS = N_DEV - 1
M_CH = 1024
K_SH = 1024
N_TOT = 8192
NC = 8
N_BLK = N_TOT // NC
N_SUB = N_BLK // 4

PIPES = ((0, +1, 0), (1, -1, N_SUB), (2, +1, 2 * N_SUB), (3, -1, 3 * N_SUB))


def kernel(x, w_mat, scale_x, scale_w):
    xb = x.astype(jnp.float8_e4m3fn)
    sc = (scale_x * scale_w).astype(jnp.float32).reshape(1, 1)

    def body(s_ref, x_any, wc_ref, wn_ref, o_ref,
             x_vmem, wc_f8, wn_f8, send_buf, recv_buf,
             copy_sem, send_sems, recv_sems, credit_sems):
        i = lax.axis_index("i")
        left = (i + N_DEV - 1) % N_DEV
        right = (i + 1) % N_DEV
        step = pl.program_id(0)
        barrier = pltpu.get_barrier_semaphore()

        def jchunk(dirn, h):
            if dirn > 0:
                return (i + N_DEV - 1 - h) % N_DEV
            return (i + 1 + h) % N_DEV

        def rdma(p, dirn, h):
            dst = right if dirn > 0 else left
            return pltpu.make_async_remote_copy(
                src_ref=send_buf.at[p, h],
                dst_ref=recv_buf.at[p, h],
                send_sem=send_sems.at[p, h],
                recv_sem=recv_sems.at[p, h],
                device_id=(dst,),
                device_id_type=pl.DeviceIdType.MESH,
            )

        def credit_to_sender(p, dirn):
            src = left if dirn > 0 else right
            pl.semaphore_signal(
                credit_sems.at[p], inc=1, device_id=(src,),
                device_id_type=pl.DeviceIdType.MESH,
            )

        def dot_rows(j, w_ref, off):
            return jnp.dot(
                x_vmem[pl.ds(j * M_CH, M_CH), :],
                w_ref[:, off:off + N_SUB],
                preferred_element_type=jnp.float32,
            )

        wc_f8[:, :] = wc_ref[:, :].astype(jnp.float8_e5m2)

        @pl.when(step <= NC - 2)
        def _cast_next():
            wn_f8[:, :] = wn_ref[:, :].astype(jnp.float8_e5m2)

        @pl.when(step == 0)
        def _init():
            cx = pltpu.make_async_copy(x_any, x_vmem, copy_sem)
            cx.start()
            for nbr in (left, right):
                pl.semaphore_signal(
                    barrier, inc=1, device_id=(nbr,),
                    device_id_type=pl.DeviceIdType.MESH,
                )
            pl.semaphore_wait(barrier, 2)
            cx.wait()
            for p, dirn, off in PIPES:
                d0 = dot_rows(jchunk(dirn, 0), wc_f8, off)
                send_buf[p, 0] = d0.astype(jnp.bfloat16)
                rdma(p, dirn, 0).start()

        for h in (1, 2):
            for p, dirn, off in PIPES:
                dh = dot_rows(jchunk(dirn, h), wc_f8, off)
                rdma(p, dirn, h - 1).wait_recv()

                @pl.when(step <= NC - 2)
                def _credit():
                    credit_to_sender(p, dirn)

                acc = dh + recv_buf[p, h - 1].astype(jnp.float32)

                @pl.when(step >= 1)
                def _reuse():
                    rdma(p, dirn, h).wait_send()

                send_buf[p, h] = acc.astype(jnp.bfloat16)

                @pl.when(step >= 1)
                def _gate():
                    pl.semaphore_wait(credit_sems.at[p], 1)

                rdma(p, dirn, h).start()

        @pl.when(step <= NC - 2)
        def _next_hop0():
            for p, dirn, off in PIPES:
                d0 = dot_rows(jchunk(dirn, 0), wn_f8, off)
                rdma(p, dirn, 0).wait_send()
                send_buf[p, 0] = d0.astype(jnp.bfloat16)
                pl.semaphore_wait(credit_sems.at[p], 1)
                rdma(p, dirn, 0).start()

        for p, dirn, off in PIPES:
            di = dot_rows(i, wc_f8, off)
            rdma(p, dirn, 2).wait_recv()

            @pl.when(step <= NC - 2)
            def _credit2():
                credit_to_sender(p, dirn)

            y = (di + recv_buf[p, 2].astype(jnp.float32)) * s_ref[0, 0]
            o_ref[:, off:off + N_SUB] = y / (
                1.0 + jnp.exp(-jnp.clip(y, -60.0, 60.0))
            )

        @pl.when(step == NC - 1)
        def _drain():
            for p, dirn, off in PIPES:
                for h in range(N_HOPS):
                    rdma(p, dirn, h).wait_send()

    return pl.pallas_call(
        body,
        grid=(NC,),
        out_shape=jax.ShapeDtypeStruct((M_CH, N_TOT), jnp.float32),
        in_specs=[
            pl.BlockSpec(memory_space=pltpu.SMEM),
            pl.BlockSpec(memory_space=pl.ANY),
            pl.BlockSpec((K_SH, N_BLK), lambda c: (0, c)),
            pl.BlockSpec((K_SH, N_BLK), lambda c: (0, jnp.minimum(c + 1, NC - 1))),
        ],
        out_specs=pl.BlockSpec((M_CH, N_BLK), lambda c: (0, c)),
        scratch_shapes=[
            pltpu.VMEM((N_DEV * M_CH, K_SH), jnp.float8_e4m3fn),
            pltpu.VMEM((K_SH, N_BLK), jnp.float8_e5m2),
            pltpu.VMEM((K_SH, N_BLK), jnp.float8_e5m2),
            pltpu.VMEM((4, N_HOPS, M_CH, N_SUB), jnp.bfloat16),
            pltpu.VMEM((4, N_HOPS, M_CH, N_SUB), jnp.bfloat16),
            pltpu.SemaphoreType.DMA,
            pltpu.SemaphoreType.DMA((4, N_HOPS)),
            pltpu.SemaphoreType.DMA((4, N_HOPS)),
            pltpu.SemaphoreType.REGULAR((4,)),
        ],
        compiler_params=pltpu.CompilerParams(
            collective_id=0,
            dimension_semantics=("arbitrary",),
            vmem_limit_bytes=50 * 1024 * 1024,
        ),
    )(sc, xb, w_mat, w_mat)


# device time: 311678 ns/iter; 2.3196x vs baseline; 1.0181x over previous
import jax
import jax.numpy as jnp
from jax import lax
from jax.experimental import pallas as pl
from jax.experimental.pallas import tpu as pltpu

N_DEV = 4
N_HOPS = N_DEV - 1
M_CH = 1024
K_SH = 1024
N_TOT = 8192
NC = 8
N_BLK = N_TOT // NC
N_SUB = N_BLK // 4

PIPES = ((0, +1, 0), (1, -1, N_SUB), (2, +1, 2 * N_SUB), (3, -1, 3 * N_SUB))


def kernel(x, w_mat, scale_x, scale_w):
    sc = (scale_x * scale_w).astype(jnp.float32).reshape(1, 1)

    def body(s_ref, x_any, wc_ref, wn_ref, o_ref,
             x_vmem, x_stage, wc_f8, wn_f8, send_buf, recv_buf,
             copy_sems, send_sems, recv_sems, credit_sems):
        i = lax.axis_index("i")
        left = (i + N_DEV - 1) % N_DEV
        right = (i + 1) % N_DEV
        step = pl.program_id(0)
        barrier = pltpu.get_barrier_semaphore()

        def jchunk(dirn, h):
            if dirn > 0:
                return (i + N_DEV - 1 - h) % N_DEV
            return (i + 1 + h) % N_DEV

        def rdma(p, dirn, h):
            dst = right if dirn > 0 else left
            return pltpu.make_async_remote_copy(
                src_ref=send_buf.at[p, h],
                dst_ref=recv_buf.at[p, h],
                send_sem=send_sems.at[p, h],
                recv_sem=recv_sems.at[p, h],
                device_id=(dst,),
                device_id_type=pl.DeviceIdType.MESH,
            )

        def credit_to_sender(p, dirn):
            src = left if dirn > 0 else right
            pl.semaphore_signal(
                credit_sems.at[p], inc=1, device_id=(src,),
                device_id_type=pl.DeviceIdType.MESH,
            )

        def dot_rows(j, w_ref, off):
            return jnp.dot(
                x_vmem[pl.ds(j * M_CH, M_CH), :],
                w_ref[:, off:off + N_SUB],
                preferred_element_type=jnp.float32,
            )

        @pl.when(step >= 1)
        def _promote():
            wc_f8[:, :] = wn_f8[:, :]

        @pl.when(step <= NC - 2)
        def _cast_next():
            wn_f8[:, :] = wn_ref[:, :].astype(jnp.float8_e5m2)

        @pl.when(step == 0)
        def _init():
            def xcopy(k):
                return pltpu.make_async_copy(
                    x_any.at[pl.ds(k * M_CH, M_CH), :],
                    x_stage.at[k % 2],
                    copy_sems.at[k % 2],
                )

            xcopy(0).start()
            xcopy(1).start()
            for nbr in (left, right):
                pl.semaphore_signal(
                    barrier, inc=1, device_id=(nbr,),
                    device_id_type=pl.DeviceIdType.MESH,
                )
            for k in range(N_DEV):
                xcopy(k).wait()
                x_vmem[pl.ds(k * M_CH, M_CH), :] = (
                    x_stage[k % 2].astype(jnp.float8_e4m3fn)
                )
                if k + 2 < N_DEV:
                    xcopy(k + 2).start()
            wc_f8[:, :] = wc_ref[:, :].astype(jnp.float8_e5m2)
            pl.semaphore_wait(barrier, 2)
            for p, dirn, off in PIPES:
                d0 = dot_rows(jchunk(dirn, 0), wc_f8, off)
                send_buf[p, 0] = d0.astype(jnp.bfloat16)
                rdma(p, dirn, 0).start()

        for h in (1, 2):
            for p, dirn, off in PIPES:
                dh = dot_rows(jchunk(dirn, h), wc_f8, off)
                rdma(p, dirn, h - 1).wait_recv()

                @pl.when(step <= NC - 2)
                def _credit():
                    credit_to_sender(p, dirn)

                acc = dh + recv_buf[p, h - 1].astype(jnp.float32)

                @pl.when(step >= 1)
                def _reuse():
                    rdma(p, dirn, h).wait_send()

                send_buf[p, h] = acc.astype(jnp.bfloat16)

                @pl.when(step >= 1)
                def _gate():
                    pl.semaphore_wait(credit_sems.at[p], 1)

                rdma(p, dirn, h).start()

        @pl.when(step <= NC - 2)
        def _next_hop0():
            for p, dirn, off in PIPES:
                d0 = dot_rows(jchunk(dirn, 0), wn_f8, off)
                rdma(p, dirn, 0).wait_send()
                send_buf[p, 0] = d0.astype(jnp.bfloat16)
                pl.semaphore_wait(credit_sems.at[p], 1)
                rdma(p, dirn, 0).start()

        for p, dirn, off in PIPES:
            di = dot_rows(i, wc_f8, off)
            rdma(p, dirn, 2).wait_recv()

            @pl.when(step <= NC - 2)
            def _credit2():
                credit_to_sender(p, dirn)

            y = (di + recv_buf[p, 2].astype(jnp.float32)) * s_ref[0, 0]
            o_ref[:, off:off + N_SUB] = y / (
                1.0 + jnp.exp(-jnp.clip(y, -60.0, 60.0))
            )

        @pl.when(step == NC - 1)
        def _drain():
            for p, dirn, off in PIPES:
                for h in range(N_HOPS):
                    rdma(p, dirn, h).wait_send()

    return pl.pallas_call(
        body,
        grid=(NC,),
        out_shape=jax.ShapeDtypeStruct((M_CH, N_TOT), jnp.float32),
        in_specs=[
            pl.BlockSpec(memory_space=pltpu.SMEM),
            pl.BlockSpec(memory_space=pl.ANY),
            pl.BlockSpec((K_SH, N_BLK), lambda c: (0, 0)),
            pl.BlockSpec((K_SH, N_BLK), lambda c: (0, jnp.minimum(c + 1, NC - 1))),
        ],
        out_specs=pl.BlockSpec((M_CH, N_BLK), lambda c: (0, c)),
        scratch_shapes=[
            pltpu.VMEM((N_DEV * M_CH, K_SH), jnp.float8_e4m3fn),
            pltpu.VMEM((2, M_CH, K_SH), jnp.float32),
            pltpu.VMEM((K_SH, N_BLK), jnp.float8_e5m2),
            pltpu.VMEM((K_SH, N_BLK), jnp.float8_e5m2),
            pltpu.VMEM((4, N_HOPS, M_CH, N_SUB), jnp.bfloat16),
            pltpu.VMEM((4, N_HOPS, M_CH, N_SUB), jnp.bfloat16),
            pltpu.SemaphoreType.DMA((2,)),
            pltpu.SemaphoreType.DMA((4, N_HOPS)),
            pltpu.SemaphoreType.DMA((4, N_HOPS)),
            pltpu.SemaphoreType.REGULAR((4,)),
        ],
        compiler_params=pltpu.CompilerParams(
            collective_id=0,
            dimension_semantics=("arbitrary",),
            vmem_limit_bytes=50 * 1024 * 1024,
        ),
    )(sc, x, w_mat, w_mat)


# device time: 308720 ns/iter; 2.3419x vs baseline; 1.0096x over previous
import jax
import jax.numpy as jnp
from jax import lax
from jax.experimental import pallas as pl
from jax.experimental.pallas import tpu as pltpu

N_DEV = 4
N_HOPS = N_DEV - 1
M_CH = 1024
K_SH = 1024
N_TOT = 8192
NC = 8
N_BLK = N_TOT // NC
N_SUB = N_BLK // 4

PIPES = ((0, +1, 0), (1, -1, N_SUB), (2, +1, 2 * N_SUB), (3, -1, 3 * N_SUB))


def kernel(x, w_mat, scale_x, scale_w):
    sc = (scale_x * scale_w).astype(jnp.float32).reshape(1, 1)

    def body(s_ref, x_any, wc_ref, wn_ref, o_ref,
             x_vmem, x_stage, wc_f8, wn_f8, send_buf, recv_buf,
             copy_sems, send_sems, recv_sems, credit_sems):
        i = lax.axis_index("i")
        left = (i + N_DEV - 1) % N_DEV
        right = (i + 1) % N_DEV
        step = pl.program_id(0)
        barrier = pltpu.get_barrier_semaphore()

        def jchunk(dirn, h):
            if dirn > 0:
                return (i + N_DEV - 1 - h) % N_DEV
            return (i + 1 + h) % N_DEV

        def rdma(p, dirn, h):
            dst = right if dirn > 0 else left
            return pltpu.make_async_remote_copy(
                src_ref=send_buf.at[p, h],
                dst_ref=recv_buf.at[p, h],
                send_sem=send_sems.at[p, h],
                recv_sem=recv_sems.at[p, h],
                device_id=(dst,),
                device_id_type=pl.DeviceIdType.MESH,
            )

        def credit_to_sender(p, dirn):
            src = left if dirn > 0 else right
            pl.semaphore_signal(
                credit_sems.at[p], inc=1, device_id=(src,),
                device_id_type=pl.DeviceIdType.MESH,
            )

        def dot_rows(j, w_ref, off):
            return jnp.dot(
                x_vmem[pl.ds(j * M_CH, M_CH), :],
                w_ref[:, off:off + N_SUB],
                preferred_element_type=jnp.float32,
            )

        @pl.when(step >= 1)
        def _promote():
            wc_f8[:, :] = wn_f8[:, :]

        @pl.when(step == 0)
        def _init():
            def xcopy(j, slot):
                return pltpu.make_async_copy(
                    x_any.at[pl.ds(j * M_CH, M_CH), :],
                    x_stage.at[slot],
                    copy_sems.at[slot],
                )

            def xland(j, slot):
                xcopy(j, slot).wait()
                x_vmem[pl.ds(j * M_CH, M_CH), :] = (
                    x_stage[slot].astype(jnp.float8_e4m3fn)
                )

            j_r = (i + 3) % N_DEV
            j_l = (i + 1) % N_DEV
            j_m = (i + 2) % N_DEV
            xcopy(j_r, 0).start()
            xcopy(j_l, 1).start()
            for nbr in (left, right):
                pl.semaphore_signal(
                    barrier, inc=1, device_id=(nbr,),
                    device_id_type=pl.DeviceIdType.MESH,
                )
            wc_f8[:, :] = wc_ref[:, :].astype(jnp.float8_e5m2)
            xland(j_r, 0)
            xcopy(j_m, 0).start()
            xland(j_l, 1)
            xcopy(i, 1).start()
            pl.semaphore_wait(barrier, 2)
            for p, dirn, off in PIPES:
                d0 = dot_rows(jchunk(dirn, 0), wc_f8, off)
                send_buf[p, 0] = d0.astype(jnp.bfloat16)
                rdma(p, dirn, 0).start()
            xland(j_m, 0)
            xland(i, 1)

        @pl.when(step <= NC - 2)
        def _cast_next():
            wn_f8[:, :] = wn_ref[:, :].astype(jnp.float8_e5m2)

        for h in (1, 2):
            for p, dirn, off in PIPES:
                dh = dot_rows(jchunk(dirn, h), wc_f8, off)
                rdma(p, dirn, h - 1).wait_recv()

                @pl.when(step <= NC - 2)
                def _credit():
                    credit_to_sender(p, dirn)

                acc = dh + recv_buf[p, h - 1].astype(jnp.float32)

                @pl.when(step >= 1)
                def _reuse():
                    rdma(p, dirn, h).wait_send()

                send_buf[p, h] = acc.astype(jnp.bfloat16)

                @pl.when(step >= 1)
                def _gate():
                    pl.semaphore_wait(credit_sems.at[p], 1)

                rdma(p, dirn, h).start()

        @pl.when(step <= NC - 2)
        def _next_hop0():
            for p, dirn, off in PIPES:
                d0 = dot_rows(jchunk(dirn, 0), wn_f8, off)
                rdma(p, dirn, 0).wait_send()
                send_buf[p, 0] = d0.astype(jnp.bfloat16)
                pl.semaphore_wait(credit_sems.at[p], 1)
                rdma(p, dirn, 0).start()

        for p, dirn, off in PIPES:
            di = dot_rows(i, wc_f8, off)
            rdma(p, dirn, 2).wait_recv()

            @pl.when(step <= NC - 2)
            def _credit2():
                credit_to_sender(p, dirn)

            y = (di + recv_buf[p, 2].astype(jnp.float32)) * s_ref[0, 0]
            o_ref[:, off:off + N_SUB] = y / (
                1.0 + jnp.exp(-jnp.clip(y, -60.0, 60.0))
            )

        @pl.when(step == NC - 1)
        def _drain():
            for p, dirn, off in PIPES:
                for h in range(N_HOPS):
                    rdma(p, dirn, h).wait_send()

    return pl.pallas_call(
        body,
        grid=(NC,),
        out_shape=jax.ShapeDtypeStruct((M_CH, N_TOT), jnp.float32),
        in_specs=[
            pl.BlockSpec(memory_space=pltpu.SMEM),
            pl.BlockSpec(memory_space=pl.ANY),
            pl.BlockSpec((K_SH, N_BLK), lambda c: (0, 0)),
            pl.BlockSpec((K_SH, N_BLK), lambda c: (0, jnp.minimum(c + 1, NC - 1))),
        ],
        out_specs=pl.BlockSpec((M_CH, N_BLK), lambda c: (0, c)),
        scratch_shapes=[
            pltpu.VMEM((N_DEV * M_CH, K_SH), jnp.float8_e4m3fn),
            pltpu.VMEM((2, M_CH, K_SH), jnp.float32),
            pltpu.VMEM((K_SH, N_BLK), jnp.float8_e5m2),
            pltpu.VMEM((K_SH, N_BLK), jnp.float8_e5m2),
            pltpu.VMEM((4, N_HOPS, M_CH, N_SUB), jnp.bfloat16),
            pltpu.VMEM((4, N_HOPS, M_CH, N_SUB), jnp.bfloat16),
            pltpu.SemaphoreType.DMA((2,)),
            pltpu.SemaphoreType.DMA((4, N_HOPS)),
            pltpu.SemaphoreType.DMA((4, N_HOPS)),
            pltpu.SemaphoreType.REGULAR((4,)),
        ],
        compiler_params=pltpu.CompilerParams(
            collective_id=0,
            dimension_semantics=("arbitrary",),
            vmem_limit_bytes=50 * 1024 * 1024,
        ),
    )(sc, x, w_mat, w_mat)
